# Initial kernel scaffold; baseline (speedup 1.0000x reference)
import jax
import jax.numpy as jnp
from jax.experimental import pallas as pl


def kernel(x, edge_index, edge_offset, W1, b1, W2, b2, W3, b3):
    raise NotImplementedError("write your pallas kernel here")


# speedup vs baseline: 7.4504x; 7.4504x over previous
import functools

import jax
import jax.numpy as jnp
from jax import lax
from jax.experimental import pallas as pl
from jax.experimental.pallas import tpu as pltpu
from jax.experimental.pallas import tpu_sc as plsc

_N = 50000
_K = 81
_NPAD = 50176
_NF = _NPAD * 10
_E = 800000
_NW = 32
_EW = 25088
_EPAD = _EW * _NW
_B = 256
_NCHUNK = _EW // _B
_JB = _B // 128
_EROWS = _EPAD // 128
_RPW = _EW // 128

_MESH = plsc.VectorSubcoreMesh(core_axis_name="c", subcore_axis_name="s")
_CP = pltpu.CompilerParams(needs_layout_passes=False,
                           use_tc_tiling_on_sc=False)


def _make_conv1():
    @functools.partial(
        pl.kernel,
        out_type=jax.ShapeDtypeStruct((2 * _NF,), jnp.float32),
        mesh=_MESH,
        compiler_params=_CP,
        scratch_types=[
            pltpu.VMEM((_NPAD,), jnp.float32),
            pltpu.VMEM((_K * 10,), jnp.float32),
            pltpu.VMEM((_JB, 128), jnp.int32),
            pltpu.VMEM((_JB, 128), jnp.int32),
            pltpu.VMEM((_JB, 128), jnp.int32),
            pltpu.VMEM((10 * _JB, 128), jnp.int32),
            pltpu.VMEM((10, _B), jnp.float32),
            pltpu.VMEM_SHARED((_NF,), jnp.float32),
            pltpu.SemaphoreType.DMA,
        ],
    )
    def conv(h_hbm, gi_hbm, si_hbm, of_hbm, w_hbm, z_hbm, out_hbm,
             h0_v, w_v, gi_v, si_v, of_v, sx_v, msg_v, out_sh, sem):
        cid = lax.axis_index("c")
        sid = lax.axis_index("s")
        wid = sid * 2 + cid
        f0 = sid * (_NF // 16)
        pltpu.sync_copy(h_hbm, h0_v)
        pltpu.sync_copy(w_hbm, w_v)
        pltpu.sync_copy(z_hbm.at[pl.ds(f0, _NF // 16)],
                        out_sh.at[pl.ds(f0, _NF // 16)])
        plsc.subcore_barrier()

        row0 = wid * _RPW

        @pl.loop(0, _NCHUNK)
        def _chunk(i):
            rr = row0 + i * _JB
            pltpu.sync_copy(gi_hbm.at[pl.ds(rr, _JB)], gi_v)
            pltpu.sync_copy(si_hbm.at[pl.ds(rr, _JB)], si_v)
            pltpu.sync_copy(of_hbm.at[pl.ds(rr, _JB)], of_v)
            for j in range(_JB):
                for l in range(8):
                    sl = pl.ds(l * 16, 16)
                    d16 = si_v[j, sl] * 10
                    for o in range(10):
                        sx_v[o * _JB + j, sl] = d16 + o
            for g in range(_B // 16):
                j, l = g // 8, g % 8
                sl = pl.ds(l * 16, 16)
                src16 = gi_v[j, sl]
                off16 = of_v[j, sl]
                hval = plsc.load_gather(h0_v, [src16])
                wb = off16 * 10
                for o in range(10):
                    w16 = plsc.load_gather(w_v, [wb + o])
                    msg_v[o, pl.ds(g * 16, 16)] = hval * w16
            cps = [pltpu.async_copy(msg_v.at[o].at[pl.ds(j * 128, 128)],
                                    out_sh.at[sx_v.at[o * _JB + j]],
                                    sem, add=True)
                   for o in range(10) for j in range(_JB)]
            for cp in cps:
                cp.wait()

        plsc.subcore_barrier()
        pltpu.sync_copy(out_sh.at[pl.ds(f0, _NF // 16)],
                        out_hbm.at[pl.ds(cid * _NF + f0, _NF // 16)])

    return conv


def _make_conv10(c_out):
    nfo = _NPAD * c_out

    @functools.partial(
        pl.kernel,
        out_type=jax.ShapeDtypeStruct((2 * nfo,), jnp.float32),
        mesh=_MESH,
        compiler_params=_CP,
        scratch_types=[
            pltpu.VMEM((_K * 10 * c_out,), jnp.float32),
            pltpu.VMEM((_JB, 128), jnp.int32),
            pltpu.VMEM((_JB, 128), jnp.int32),
            pltpu.VMEM((_JB, 128), jnp.int32),
            pltpu.VMEM((10 * _JB, 128), jnp.int32),
            pltpu.VMEM((c_out * _JB, 128), jnp.int32),
            pltpu.VMEM((10, _B), jnp.float32),
            pltpu.VMEM((c_out, _B), jnp.float32),
            pltpu.VMEM_SHARED((_NF,), jnp.float32),
            pltpu.VMEM_SHARED((nfo,), jnp.float32),
            pltpu.SemaphoreType.DMA,
            pltpu.SemaphoreType.DMA,
        ],
    )
    def conv(h_hbm, gi_hbm, si_hbm, of_hbm, w_hbm, z_hbm, out_hbm,
             w_v, gi_v, si_v, of_v, gx_v, sx_v, hg_v, msg_v, h_sh, out_sh,
             gsem, ssem):
        cid = lax.axis_index("c")
        sid = lax.axis_index("s")
        wid = sid * 2 + cid
        f0 = sid * (_NF // 16)
        o0 = sid * (nfo // 16)
        pltpu.sync_copy(h_hbm.at[pl.ds(f0, _NF // 16)],
                        h_sh.at[pl.ds(f0, _NF // 16)])
        pltpu.sync_copy(z_hbm.at[pl.ds(o0, nfo // 16)],
                        out_sh.at[pl.ds(o0, nfo // 16)])
        pltpu.sync_copy(w_hbm, w_v)
        plsc.subcore_barrier()

        row0 = wid * _RPW

        @pl.loop(0, _NCHUNK)
        def _chunk(i):
            rr = row0 + i * _JB
            pltpu.sync_copy(gi_hbm.at[pl.ds(rr, _JB)], gi_v)
            pltpu.sync_copy(si_hbm.at[pl.ds(rr, _JB)], si_v)
            pltpu.sync_copy(of_hbm.at[pl.ds(rr, _JB)], of_v)
            for j in range(_JB):
                for l in range(8):
                    sl = pl.ds(l * 16, 16)
                    s16 = gi_v[j, sl] * 10
                    for c in range(10):
                        gx_v[c * _JB + j, sl] = s16 + c
                    d16 = si_v[j, sl] * c_out
                    for o in range(c_out):
                        sx_v[o * _JB + j, sl] = d16 + o
            gps = [pltpu.async_copy(h_sh.at[gx_v.at[c * _JB + j]],
                                    hg_v.at[c].at[pl.ds(j * 128, 128)],
                                    gsem)
                   for c in range(10) for j in range(_JB)]
            for cp in gps:
                cp.wait()
            for g in range(_B // 16):
                sl = pl.ds(g * 16, 16)
                off16 = of_v[g // 8, pl.ds((g % 8) * 16, 16)]
                wb = off16 * (10 * c_out)
                hcol = [hg_v[c, sl] for c in range(10)]
                for o in range(c_out):
                    acc = hcol[0] * plsc.load_gather(w_v, [wb + o])
                    for c2 in range(1, 10):
                        acc = acc + hcol[c2] * plsc.load_gather(
                            w_v, [wb + (c2 * c_out + o)])
                    msg_v[o, sl] = acc
            sps = [pltpu.async_copy(msg_v.at[o].at[pl.ds(j * 128, 128)],
                                    out_sh.at[sx_v.at[o * _JB + j]],
                                    ssem, add=True)
                   for o in range(c_out) for j in range(_JB)]
            for cp in sps:
                cp.wait()

        plsc.subcore_barrier()
        pltpu.sync_copy(out_sh.at[pl.ds(o0, nfo // 16)],
                        out_hbm.at[pl.ds(cid * nfo + o0, nfo // 16)])

    return conv


_conv1 = _make_conv1()
_conv10_5 = _make_conv10(5)
_conv10_1 = _make_conv10(1)


def kernel(x, edge_index, edge_offset, W1, b1, W2, b2, W3, b3):
    h0 = jax.nn.relu(x[:, 0].astype(jnp.float32))
    h0p = jnp.pad(h0, (0, _NPAD - _N))
    src = edge_index[0].astype(jnp.int32)
    dst = edge_index[1].astype(jnp.int32)
    off = edge_offset.astype(jnp.int32)
    npad_e = _EPAD - _E
    srcp = jnp.concatenate(
        [src, jnp.full((npad_e,), _N, jnp.int32)]).reshape(_EROWS, 128)
    dstp = jnp.concatenate(
        [dst, jnp.full((npad_e,), _N, jnp.int32)]).reshape(_EROWS, 128)
    offp = jnp.concatenate(
        [off, jnp.zeros((npad_e,), jnp.int32)]).reshape(_EROWS, 128)
    offm = (_K - 1) - offp
    z10 = jnp.zeros((_NF,), jnp.float32)
    z1 = jnp.zeros((_NPAD,), jnp.float32)
    w1f = W1.reshape(-1).astype(jnp.float32)
    w2 = W2.astype(jnp.float32)
    w2a = w2[:, :, :5].reshape(-1)
    w2b = w2[:, :, 5:].reshape(-1)
    w3f = W3.reshape(-1).astype(jnp.float32)
    z5 = z10[:_NPAD * 5]

    def stack(gi, si, of):
        o1 = _conv1(h0p, gi, si, of, w1f, z10)
        h1 = jax.nn.relu((o1[:_NF] + o1[_NF:]).reshape(_NPAD, 10) + b1)
        h1f = h1.reshape(-1)
        o2a = _conv10_5(h1f, gi, si, of, w2a, z5)
        o2b = _conv10_5(h1f, gi, si, of, w2b, z5)
        nf5 = _NPAD * 5
        h2a = (o2a[:nf5] + o2a[nf5:]).reshape(_NPAD, 5)
        h2b = (o2b[:nf5] + o2b[nf5:]).reshape(_NPAD, 5)
        h2 = jax.nn.relu(jnp.concatenate([h2a, h2b], axis=1) + b2)
        o3 = _conv10_1(h2.reshape(-1), gi, si, of, w3f, z1)
        return jax.nn.relu(o3[:_NPAD] + o3[_NPAD:] + b3)

    out_f = stack(srcp, dstp, offp)
    out_b = stack(dstp, srcp, offm)
    return (out_f + out_b)[:_N, None]


# speedup vs baseline: 9.9261x; 1.3323x over previous
import functools

import jax
import jax.numpy as jnp
from jax import lax
from jax.experimental import pallas as pl
from jax.experimental.pallas import tpu as pltpu
from jax.experimental.pallas import tpu_sc as plsc

_N = 50000
_K = 81
_NPAD = 50176
_NF = _NPAD * 10
_E = 800000
_NW = 32
_EW = 25088
_EPAD = _EW * _NW
_B = 256
_JB = _B // 128
_NITER = _EW // (2 * _B)
_EROWS = _EPAD // 128
_RPW = _EW // 128

_MESH = plsc.VectorSubcoreMesh(core_axis_name="c", subcore_axis_name="s")
_CP = pltpu.CompilerParams(needs_layout_passes=False,
                           use_tc_tiling_on_sc=False)


def _make_conv1():
    @functools.partial(
        pl.kernel,
        out_type=jax.ShapeDtypeStruct((2 * _NF,), jnp.float32),
        mesh=_MESH,
        compiler_params=_CP,
        scratch_types=[
            pltpu.VMEM((_NPAD,), jnp.float32),
            pltpu.VMEM((_K * 10,), jnp.float32),
            pltpu.VMEM((2 * _JB, 128), jnp.int32),
            pltpu.VMEM((2 * _JB, 128), jnp.int32),
            pltpu.VMEM((2 * _JB, 128), jnp.int32),
            pltpu.VMEM((2 * 10 * _JB, 128), jnp.int32),
            pltpu.VMEM((2 * 10, _B), jnp.float32),
            pltpu.VMEM_SHARED((_NF,), jnp.float32),
            pltpu.SemaphoreType.DMA,
            pltpu.SemaphoreType.DMA,
        ],
    )
    def conv(h_hbm, gi_hbm, si_hbm, of_hbm, w_hbm, z_hbm, out_hbm,
             h0_v, w_v, gi_v, si_v, of_v, sx_v, msg_v, out_sh, isem, ssem):
        cid = lax.axis_index("c")
        sid = lax.axis_index("s")
        wid = sid * 2 + cid
        f0 = sid * (_NF // 16)
        pltpu.sync_copy(h_hbm, h0_v)
        pltpu.sync_copy(w_hbm, w_v)
        pltpu.sync_copy(z_hbm.at[pl.ds(f0, _NF // 16)],
                        out_sh.at[pl.ds(f0, _NF // 16)])
        plsc.subcore_barrier()

        row0 = wid * _RPW
        rpi = 2 * _JB

        def fire_idx(i):
            rr = row0 + i * rpi
            pltpu.async_copy(gi_hbm.at[pl.ds(rr, rpi)], gi_v, isem)
            pltpu.async_copy(si_hbm.at[pl.ds(rr, rpi)], si_v, isem)
            pltpu.async_copy(of_hbm.at[pl.ds(rr, rpi)], of_v, isem)

        def drain_scatters():
            for r in range(2 * 10):
                for j in range(_JB):
                    pltpu.make_async_copy(
                        msg_v.at[r].at[pl.ds(j * 128, 128)],
                        out_sh.at[sx_v.at[r * _JB + j]], ssem).wait()

        fire_idx(0)

        @pl.loop(0, _NITER)
        def _step(i):
            @pl.when(i > 0)
            def _():
                drain_scatters()

            rr = row0 + i * rpi
            pltpu.make_async_copy(gi_hbm.at[pl.ds(rr, rpi)], gi_v, isem).wait()
            pltpu.make_async_copy(si_hbm.at[pl.ds(rr, rpi)], si_v, isem).wait()
            pltpu.make_async_copy(of_hbm.at[pl.ds(rr, rpi)], of_v, isem).wait()

            for p in range(2):
                for j in range(_JB):
                    for l in range(8):
                        sl = pl.ds(l * 16, 16)
                        d16 = si_v[p * _JB + j, sl] * 10
                        for o in range(10):
                            sx_v[(p * 10 + o) * _JB + j, sl] = d16 + o
                for g in range(_B // 16):
                    j, l = g // 8, g % 8
                    sl = pl.ds(l * 16, 16)
                    src16 = gi_v[p * _JB + j, sl]
                    off16 = of_v[p * _JB + j, sl]
                    hval = plsc.load_gather(h0_v, [src16])
                    wb = off16 * 10
                    for o in range(10):
                        w16 = plsc.load_gather(w_v, [wb + o])
                        msg_v[p * 10 + o, pl.ds(g * 16, 16)] = hval * w16
                for o in range(10):
                    for j in range(_JB):
                        pltpu.async_copy(
                            msg_v.at[(p * 10 + o)].at[pl.ds(j * 128, 128)],
                            out_sh.at[sx_v.at[(p * 10 + o) * _JB + j]],
                            ssem, add=True)

            @pl.when(i < _NITER - 1)
            def _():
                fire_idx(i + 1)

        drain_scatters()
        plsc.subcore_barrier()
        pltpu.sync_copy(out_sh.at[pl.ds(f0, _NF // 16)],
                        out_hbm.at[pl.ds(cid * _NF + f0, _NF // 16)])

    return conv


def _make_conv10(c_out):
    nfo = _NPAD * c_out

    @functools.partial(
        pl.kernel,
        out_type=jax.ShapeDtypeStruct((2 * nfo,), jnp.float32),
        mesh=_MESH,
        compiler_params=_CP,
        scratch_types=[
            pltpu.VMEM((_K * 10 * c_out,), jnp.float32),
            pltpu.VMEM((2 * _JB, 128), jnp.int32),
            pltpu.VMEM((2 * _JB, 128), jnp.int32),
            pltpu.VMEM((2 * _JB, 128), jnp.int32),
            pltpu.VMEM((2 * 10 * _JB, 128), jnp.int32),
            pltpu.VMEM((2 * c_out * _JB, 128), jnp.int32),
            pltpu.VMEM((2 * 10, _B), jnp.float32),
            pltpu.VMEM((2 * c_out, _B), jnp.float32),
            pltpu.VMEM_SHARED((_NF,), jnp.float32),
            pltpu.VMEM_SHARED((nfo,), jnp.float32),
            pltpu.SemaphoreType.DMA,
            pltpu.SemaphoreType.DMA,
            pltpu.SemaphoreType.DMA,
            pltpu.SemaphoreType.DMA,
        ],
    )
    def conv(h_hbm, gi_hbm, si_hbm, of_hbm, w_hbm, z_hbm, out_hbm,
             w_v, gi_v, si_v, of_v, gx_v, sx_v, hg_v, msg_v, h_sh, out_sh,
             isem, gsem0, gsem1, ssem):
        cid = lax.axis_index("c")
        sid = lax.axis_index("s")
        wid = sid * 2 + cid
        f0 = sid * (_NF // 16)
        o0 = sid * (nfo // 16)
        pltpu.sync_copy(h_hbm.at[pl.ds(f0, _NF // 16)],
                        h_sh.at[pl.ds(f0, _NF // 16)])
        pltpu.sync_copy(z_hbm.at[pl.ds(o0, nfo // 16)],
                        out_sh.at[pl.ds(o0, nfo // 16)])
        pltpu.sync_copy(w_hbm, w_v)
        plsc.subcore_barrier()

        row0 = wid * _RPW
        rpi = 2 * _JB
        gsems = (gsem0, gsem1)

        def fire_idx(i):
            rr = row0 + i * rpi
            pltpu.async_copy(gi_hbm.at[pl.ds(rr, rpi)], gi_v, isem)
            pltpu.async_copy(si_hbm.at[pl.ds(rr, rpi)], si_v, isem)
            pltpu.async_copy(of_hbm.at[pl.ds(rr, rpi)], of_v, isem)

        def drain_scatters():
            for r in range(2 * c_out):
                for j in range(_JB):
                    pltpu.make_async_copy(
                        msg_v.at[r].at[pl.ds(j * 128, 128)],
                        out_sh.at[sx_v.at[r * _JB + j]], ssem).wait()

        fire_idx(0)

        @pl.loop(0, _NITER)
        def _step(i):
            @pl.when(i > 0)
            def _():
                drain_scatters()

            rr = row0 + i * rpi
            pltpu.make_async_copy(gi_hbm.at[pl.ds(rr, rpi)], gi_v, isem).wait()
            pltpu.make_async_copy(si_hbm.at[pl.ds(rr, rpi)], si_v, isem).wait()
            pltpu.make_async_copy(of_hbm.at[pl.ds(rr, rpi)], of_v, isem).wait()

            gps = [[], []]
            for p in range(2):
                for j in range(_JB):
                    for l in range(8):
                        sl = pl.ds(l * 16, 16)
                        s16 = gi_v[p * _JB + j, sl] * 10
                        for c in range(10):
                            gx_v[(p * 10 + c) * _JB + j, sl] = s16 + c
                        d16 = si_v[p * _JB + j, sl] * c_out
                        for o in range(c_out):
                            sx_v[(p * c_out + o) * _JB + j, sl] = d16 + o
                gps[p] = [
                    pltpu.async_copy(h_sh.at[gx_v.at[(p * 10 + c) * _JB + j]],
                                     hg_v.at[p * 10 + c].at[pl.ds(j * 128,
                                                                  128)],
                                     gsems[p])
                    for c in range(10) for j in range(_JB)]

            for p in range(2):
                for cp in gps[p]:
                    cp.wait()
                for g in range(_B // 16):
                    sl = pl.ds(g * 16, 16)
                    off16 = of_v[p * _JB + g // 8, pl.ds((g % 8) * 16, 16)]
                    wb = off16 * (10 * c_out)
                    hcol = [hg_v[p * 10 + c, sl] for c in range(10)]
                    for o in range(c_out):
                        acc = hcol[0] * plsc.load_gather(w_v, [wb + o])
                        for c2 in range(1, 10):
                            acc = acc + hcol[c2] * plsc.load_gather(
                                w_v, [wb + (c2 * c_out + o)])
                        msg_v[p * c_out + o, sl] = acc
                for o in range(c_out):
                    for j in range(_JB):
                        pltpu.async_copy(
                            msg_v.at[p * c_out + o].at[pl.ds(j * 128, 128)],
                            out_sh.at[sx_v.at[(p * c_out + o) * _JB + j]],
                            ssem, add=True)

            @pl.when(i < _NITER - 1)
            def _():
                fire_idx(i + 1)

        drain_scatters()
        plsc.subcore_barrier()
        pltpu.sync_copy(out_sh.at[pl.ds(o0, nfo // 16)],
                        out_hbm.at[pl.ds(cid * nfo + o0, nfo // 16)])

    return conv


_conv1 = _make_conv1()
_conv10_5 = _make_conv10(5)
_conv10_1 = _make_conv10(1)


def kernel(x, edge_index, edge_offset, W1, b1, W2, b2, W3, b3):
    h0 = jax.nn.relu(x[:, 0].astype(jnp.float32))
    h0p = jnp.pad(h0, (0, _NPAD - _N))
    src = edge_index[0].astype(jnp.int32)
    dst = edge_index[1].astype(jnp.int32)
    off = edge_offset.astype(jnp.int32)
    npad_e = _EPAD - _E
    srcp = jnp.concatenate(
        [src, jnp.full((npad_e,), _N, jnp.int32)]).reshape(_EROWS, 128)
    dstp = jnp.concatenate(
        [dst, jnp.full((npad_e,), _N, jnp.int32)]).reshape(_EROWS, 128)
    offp = jnp.concatenate(
        [off, jnp.zeros((npad_e,), jnp.int32)]).reshape(_EROWS, 128)
    offm = (_K - 1) - offp
    z10 = jnp.zeros((_NF,), jnp.float32)
    z1 = jnp.zeros((_NPAD,), jnp.float32)
    w1f = W1.reshape(-1).astype(jnp.float32)
    w2 = W2.astype(jnp.float32)
    w2a = w2[:, :, :5].reshape(-1)
    w2b = w2[:, :, 5:].reshape(-1)
    w3f = W3.reshape(-1).astype(jnp.float32)
    z5 = z10[:_NPAD * 5]

    def stack(gi, si, of):
        o1 = _conv1(h0p, gi, si, of, w1f, z10)
        h1 = jax.nn.relu((o1[:_NF] + o1[_NF:]).reshape(_NPAD, 10) + b1)
        h1f = h1.reshape(-1)
        o2a = _conv10_5(h1f, gi, si, of, w2a, z5)
        o2b = _conv10_5(h1f, gi, si, of, w2b, z5)
        nf5 = _NPAD * 5
        h2a = (o2a[:nf5] + o2a[nf5:]).reshape(_NPAD, 5)
        h2b = (o2b[:nf5] + o2b[nf5:]).reshape(_NPAD, 5)
        h2 = jax.nn.relu(jnp.concatenate([h2a, h2b], axis=1) + b2)
        o3 = _conv10_1(h2.reshape(-1), gi, si, of, w3f, z1)
        return jax.nn.relu(o3[:_NPAD] + o3[_NPAD:] + b3)

    out_f = stack(srcp, dstp, offp)
    out_b = stack(dstp, srcp, offm)
    return (out_f + out_b)[:_N, None]


# speedup vs baseline: 11.8354x; 1.1924x over previous
import functools

import jax
import jax.numpy as jnp
from jax import lax
from jax.experimental import pallas as pl
from jax.experimental.pallas import tpu as pltpu
from jax.experimental.pallas import tpu_sc as plsc

_N = 50000
_K = 81
_NPAD = 50176
_NF = _NPAD * 10
_E = 800000
_NW = 32
_EW = 25088
_EPAD = _EW * _NW
_B = 256
_JB = _B // 128
_NITER = _EW // (2 * _B)
_EROWS = _EPAD // 128
_RPW = _EW // 128

_MESH = plsc.VectorSubcoreMesh(core_axis_name="c", subcore_axis_name="s")
_CP = pltpu.CompilerParams(needs_layout_passes=False,
                           use_tc_tiling_on_sc=False)


def _make_conv1():
    @functools.partial(
        pl.kernel,
        out_type=jax.ShapeDtypeStruct((2 * _NF,), jnp.float32),
        mesh=_MESH,
        compiler_params=_CP,
        scratch_types=[
            pltpu.VMEM((_NPAD,), jnp.float32),
            pltpu.VMEM((_K * 10,), jnp.float32),
            pltpu.VMEM((2 * _JB, 128), jnp.int32),
            pltpu.VMEM((2 * _JB, 128), jnp.int32),
            pltpu.VMEM((2 * _JB, 128), jnp.int32),
            pltpu.VMEM((2 * 10 * _JB, 128), jnp.int32),
            pltpu.VMEM((2 * 10, _B), jnp.float32),
            pltpu.VMEM_SHARED((_NF,), jnp.float32),
            pltpu.SemaphoreType.DMA,
            pltpu.SemaphoreType.DMA,
        ],
    )
    def conv(h_hbm, gi_hbm, si_hbm, of_hbm, w_hbm, z_hbm, out_hbm,
             h0_v, w_v, gi_v, si_v, of_v, sx_v, msg_v, out_sh, isem, ssem):
        cid = lax.axis_index("c")
        sid = lax.axis_index("s")
        wid = sid * 2 + cid
        f0 = sid * (_NF // 16)
        pltpu.sync_copy(h_hbm, h0_v)
        pltpu.sync_copy(w_hbm, w_v)
        pltpu.sync_copy(z_hbm.at[pl.ds(f0, _NF // 16)],
                        out_sh.at[pl.ds(f0, _NF // 16)])
        plsc.subcore_barrier()

        row0 = wid * _RPW
        rpi = 2 * _JB

        def fire_idx(i):
            rr = row0 + i * rpi
            pltpu.async_copy(gi_hbm.at[pl.ds(rr, rpi)], gi_v, isem)
            pltpu.async_copy(si_hbm.at[pl.ds(rr, rpi)], si_v, isem)
            pltpu.async_copy(of_hbm.at[pl.ds(rr, rpi)], of_v, isem)

        def drain_scatters():
            for r in range(2 * 10):
                for j in range(_JB):
                    pltpu.make_async_copy(
                        msg_v.at[r].at[pl.ds(j * 128, 128)],
                        out_sh.at[sx_v.at[r * _JB + j]], ssem).wait()

        fire_idx(0)

        @pl.loop(0, _NITER)
        def _step(i):
            @pl.when(i > 0)
            def _():
                drain_scatters()

            rr = row0 + i * rpi
            pltpu.make_async_copy(gi_hbm.at[pl.ds(rr, rpi)], gi_v, isem).wait()
            pltpu.make_async_copy(si_hbm.at[pl.ds(rr, rpi)], si_v, isem).wait()
            pltpu.make_async_copy(of_hbm.at[pl.ds(rr, rpi)], of_v, isem).wait()

            for p in range(2):
                for j in range(_JB):
                    for l in range(8):
                        sl = pl.ds(l * 16, 16)
                        d16 = si_v[p * _JB + j, sl]
                        for o in range(10):
                            sx_v[(p * 10 + o) * _JB + j, sl] = d16 + o * _NPAD
                for g in range(_B // 16):
                    j, l = g // 8, g % 8
                    sl = pl.ds(l * 16, 16)
                    src16 = gi_v[p * _JB + j, sl]
                    off16 = of_v[p * _JB + j, sl]
                    hval = plsc.load_gather(h0_v, [src16])
                    wb = off16 * 10
                    for o in range(10):
                        w16 = plsc.load_gather(w_v, [wb + o])
                        msg_v[p * 10 + o, pl.ds(g * 16, 16)] = hval * w16
                for o in range(10):
                    for j in range(_JB):
                        pltpu.async_copy(
                            msg_v.at[(p * 10 + o)].at[pl.ds(j * 128, 128)],
                            out_sh.at[sx_v.at[(p * 10 + o) * _JB + j]],
                            ssem, add=True)

            @pl.when(i < _NITER - 1)
            def _():
                fire_idx(i + 1)

        drain_scatters()
        plsc.subcore_barrier()
        pltpu.sync_copy(out_sh.at[pl.ds(f0, _NF // 16)],
                        out_hbm.at[pl.ds(cid * _NF + f0, _NF // 16)])

    return conv


def _make_conv10(c_out):
    nfo = _NPAD * c_out

    @functools.partial(
        pl.kernel,
        out_type=jax.ShapeDtypeStruct((2 * nfo,), jnp.float32),
        mesh=_MESH,
        compiler_params=_CP,
        scratch_types=[
            pltpu.VMEM((_K * 10 * c_out,), jnp.float32),
            pltpu.VMEM((2 * _JB, 128), jnp.int32),
            pltpu.VMEM((2 * _JB, 128), jnp.int32),
            pltpu.VMEM((2 * _JB, 128), jnp.int32),
            pltpu.VMEM((2 * 10 * _JB, 128), jnp.int32),
            pltpu.VMEM((2 * c_out * _JB, 128), jnp.int32),
            pltpu.VMEM((2 * 10, _B), jnp.float32),
            pltpu.VMEM((2 * c_out, _B), jnp.float32),
            pltpu.VMEM_SHARED((_NF,), jnp.float32),
            pltpu.VMEM_SHARED((nfo,), jnp.float32),
            pltpu.SemaphoreType.DMA,
            pltpu.SemaphoreType.DMA,
            pltpu.SemaphoreType.DMA,
            pltpu.SemaphoreType.DMA,
        ],
    )
    def conv(h_hbm, gi_hbm, si_hbm, of_hbm, w_hbm, z_hbm, out_hbm,
             w_v, gi_v, si_v, of_v, gx_v, sx_v, hg_v, msg_v, h_sh, out_sh,
             isem, gsem0, gsem1, ssem):
        cid = lax.axis_index("c")
        sid = lax.axis_index("s")
        wid = sid * 2 + cid
        f0 = sid * (_NF // 16)
        o0 = sid * (nfo // 16)
        pltpu.sync_copy(h_hbm.at[pl.ds(f0, _NF // 16)],
                        h_sh.at[pl.ds(f0, _NF // 16)])
        pltpu.sync_copy(z_hbm.at[pl.ds(o0, nfo // 16)],
                        out_sh.at[pl.ds(o0, nfo // 16)])
        pltpu.sync_copy(w_hbm, w_v)
        plsc.subcore_barrier()

        row0 = wid * _RPW
        rpi = 2 * _JB
        gsems = (gsem0, gsem1)

        def fire_idx(i):
            rr = row0 + i * rpi
            pltpu.async_copy(gi_hbm.at[pl.ds(rr, rpi)], gi_v, isem)
            pltpu.async_copy(si_hbm.at[pl.ds(rr, rpi)], si_v, isem)
            pltpu.async_copy(of_hbm.at[pl.ds(rr, rpi)], of_v, isem)

        def drain_scatters():
            for r in range(2 * c_out):
                for j in range(_JB):
                    pltpu.make_async_copy(
                        msg_v.at[r].at[pl.ds(j * 128, 128)],
                        out_sh.at[sx_v.at[r * _JB + j]], ssem).wait()

        fire_idx(0)

        @pl.loop(0, _NITER)
        def _step(i):
            @pl.when(i > 0)
            def _():
                drain_scatters()

            rr = row0 + i * rpi
            pltpu.make_async_copy(gi_hbm.at[pl.ds(rr, rpi)], gi_v, isem).wait()
            pltpu.make_async_copy(si_hbm.at[pl.ds(rr, rpi)], si_v, isem).wait()
            pltpu.make_async_copy(of_hbm.at[pl.ds(rr, rpi)], of_v, isem).wait()

            gps = [[], []]
            for p in range(2):
                for j in range(_JB):
                    for l in range(8):
                        sl = pl.ds(l * 16, 16)
                        s16 = gi_v[p * _JB + j, sl]
                        for c in range(10):
                            gx_v[(p * 10 + c) * _JB + j, sl] = s16 + c * _NPAD
                        d16 = si_v[p * _JB + j, sl]
                        for o in range(c_out):
                            sx_v[(p * c_out + o) * _JB + j, sl] = (
                                d16 + o * _NPAD)
                gps[p] = [
                    pltpu.async_copy(h_sh.at[gx_v.at[(p * 10 + c) * _JB + j]],
                                     hg_v.at[p * 10 + c].at[pl.ds(j * 128,
                                                                  128)],
                                     gsems[p])
                    for c in range(10) for j in range(_JB)]

            for p in range(2):
                for cp in gps[p]:
                    cp.wait()
                for g in range(_B // 16):
                    sl = pl.ds(g * 16, 16)
                    off16 = of_v[p * _JB + g // 8, pl.ds((g % 8) * 16, 16)]
                    wb = off16 * (10 * c_out)
                    hcol = [hg_v[p * 10 + c, sl] for c in range(10)]
                    for o in range(c_out):
                        acc = hcol[0] * plsc.load_gather(w_v, [wb + o])
                        for c2 in range(1, 10):
                            acc = acc + hcol[c2] * plsc.load_gather(
                                w_v, [wb + (c2 * c_out + o)])
                        msg_v[p * c_out + o, sl] = acc
                for o in range(c_out):
                    for j in range(_JB):
                        pltpu.async_copy(
                            msg_v.at[p * c_out + o].at[pl.ds(j * 128, 128)],
                            out_sh.at[sx_v.at[(p * c_out + o) * _JB + j]],
                            ssem, add=True)

            @pl.when(i < _NITER - 1)
            def _():
                fire_idx(i + 1)

        drain_scatters()
        plsc.subcore_barrier()
        pltpu.sync_copy(out_sh.at[pl.ds(o0, nfo // 16)],
                        out_hbm.at[pl.ds(cid * nfo + o0, nfo // 16)])

    return conv


_conv1 = _make_conv1()
_conv10_5 = _make_conv10(5)
_conv10_1 = _make_conv10(1)


def kernel(x, edge_index, edge_offset, W1, b1, W2, b2, W3, b3):
    h0 = jax.nn.relu(x[:, 0].astype(jnp.float32))
    h0p = jnp.pad(h0, (0, _NPAD - _N))
    src = edge_index[0].astype(jnp.int32)
    dst = edge_index[1].astype(jnp.int32)
    off = edge_offset.astype(jnp.int32)
    npad_e = _EPAD - _E
    srcp = jnp.concatenate(
        [src, jnp.full((npad_e,), _N, jnp.int32)]).reshape(_EROWS, 128)
    dstp = jnp.concatenate(
        [dst, jnp.full((npad_e,), _N, jnp.int32)]).reshape(_EROWS, 128)
    offp = jnp.concatenate(
        [off, jnp.zeros((npad_e,), jnp.int32)]).reshape(_EROWS, 128)
    offm = (_K - 1) - offp
    z10 = jnp.zeros((_NF,), jnp.float32)
    z1 = jnp.zeros((_NPAD,), jnp.float32)
    w1f = W1.reshape(-1).astype(jnp.float32)
    w2 = W2.astype(jnp.float32)
    w2a = w2[:, :, :5].reshape(-1)
    w2b = w2[:, :, 5:].reshape(-1)
    w3f = W3.reshape(-1).astype(jnp.float32)
    z5 = z10[:_NPAD * 5]

    def stack(gi, si, of):
        o1 = _conv1(h0p, gi, si, of, w1f, z10)
        h1 = jax.nn.relu((o1[:_NF] + o1[_NF:]).reshape(10, _NPAD) + b1[:, None])
        h1f = h1.reshape(-1)
        o2a = _conv10_5(h1f, gi, si, of, w2a, z5)
        o2b = _conv10_5(h1f, gi, si, of, w2b, z5)
        nf5 = _NPAD * 5
        h2a = (o2a[:nf5] + o2a[nf5:]).reshape(5, _NPAD)
        h2b = (o2b[:nf5] + o2b[nf5:]).reshape(5, _NPAD)
        h2 = jax.nn.relu(jnp.concatenate([h2a, h2b], axis=0) + b2[:, None])
        o3 = _conv10_1(h2.reshape(-1), gi, si, of, w3f, z1)
        return jax.nn.relu(o3[:_NPAD] + o3[_NPAD:] + b3)

    out_f = stack(srcp, dstp, offp)
    out_b = stack(dstp, srcp, offm)
    return (out_f + out_b)[:_N, None]


# speedup vs baseline: 20.5662x; 1.7377x over previous
import functools

import jax
import jax.numpy as jnp
from jax import lax
from jax.experimental import pallas as pl
from jax.experimental.pallas import tpu as pltpu
from jax.experimental.pallas import tpu_sc as plsc

_N = 50000
_K = 81
_NPAD = 50176
_NF = _NPAD * 10
_E = 800000
_NW = 32
_EW = 25088
_EPAD = _EW * _NW
_B = 256
_JB = _B // 128
_NITER = _EW // (2 * _B)
_EROWS = _EPAD // 128
_RPW = _EW // 128

_MESH = plsc.VectorSubcoreMesh(core_axis_name="c", subcore_axis_name="s")
_CP = pltpu.CompilerParams(needs_layout_passes=False,
                           use_tc_tiling_on_sc=False)


def _make_conv1():
    @functools.partial(
        pl.kernel,
        out_type=jax.ShapeDtypeStruct((2 * _NF,), jnp.float32),
        mesh=_MESH,
        compiler_params=_CP,
        scratch_types=[
            pltpu.VMEM((_NPAD,), jnp.float32),
            pltpu.VMEM((_K * 10,), jnp.float32),
            pltpu.VMEM((2 * _JB, 128), jnp.int32),
            pltpu.VMEM((2 * _JB, 128), jnp.int32),
            pltpu.VMEM((2 * _JB, 128), jnp.int32),
            pltpu.VMEM((2 * 10 * _JB, 128), jnp.int32),
            pltpu.VMEM((2 * 10, _B), jnp.float32),
            pltpu.VMEM_SHARED((_NF,), jnp.float32),
            pltpu.SemaphoreType.DMA,
            pltpu.SemaphoreType.DMA,
        ],
    )
    def conv(h_hbm, gi_hbm, si_hbm, of_hbm, w_hbm, z_hbm, out_hbm,
             h0_v, w_v, gi_v, si_v, of_v, sx_v, msg_v, out_sh, isem, ssem):
        cid = lax.axis_index("c")
        sid = lax.axis_index("s")
        wid = sid * 2 + cid
        f0 = sid * (_NF // 16)
        pltpu.sync_copy(h_hbm, h0_v)
        pltpu.sync_copy(w_hbm, w_v)
        pltpu.sync_copy(z_hbm.at[pl.ds(f0, _NF // 16)],
                        out_sh.at[pl.ds(f0, _NF // 16)])
        plsc.subcore_barrier()

        row0 = wid * _RPW
        rpi = 2 * _JB

        def fire_idx(i):
            rr = row0 + i * rpi
            pltpu.async_copy(gi_hbm.at[pl.ds(rr, rpi)], gi_v, isem)
            pltpu.async_copy(si_hbm.at[pl.ds(rr, rpi)], si_v, isem)
            pltpu.async_copy(of_hbm.at[pl.ds(rr, rpi)], of_v, isem)

        def drain_scatters():
            for r in range(2 * 10):
                for j in range(_JB):
                    pltpu.make_async_copy(
                        msg_v.at[r].at[pl.ds(j * 128, 128)],
                        out_sh.at[sx_v.at[r * _JB + j]], ssem).wait()

        fire_idx(0)

        @pl.loop(0, _NITER)
        def _step(i):
            @pl.when(i > 0)
            def _():
                drain_scatters()

            rr = row0 + i * rpi
            pltpu.make_async_copy(gi_hbm.at[pl.ds(rr, rpi)], gi_v, isem).wait()
            pltpu.make_async_copy(si_hbm.at[pl.ds(rr, rpi)], si_v, isem).wait()
            pltpu.make_async_copy(of_hbm.at[pl.ds(rr, rpi)], of_v, isem).wait()

            for p in range(2):
                for j in range(_JB):
                    for l in range(8):
                        sl = pl.ds(l * 16, 16)
                        d16 = si_v[p * _JB + j, sl]
                        for o in range(10):
                            sx_v[(p * 10 + o) * _JB + j, sl] = d16 + o * _NPAD
                for g in range(_B // 16):
                    j, l = g // 8, g % 8
                    sl = pl.ds(l * 16, 16)
                    src16 = gi_v[p * _JB + j, sl]
                    off16 = of_v[p * _JB + j, sl]
                    hval = plsc.load_gather(h0_v, [src16])
                    wb = off16 * 10
                    for o in range(10):
                        w16 = plsc.load_gather(w_v, [wb + o])
                        msg_v[p * 10 + o, pl.ds(g * 16, 16)] = hval * w16
                for o in range(10):
                    for j in range(_JB):
                        pltpu.async_copy(
                            msg_v.at[(p * 10 + o)].at[pl.ds(j * 128, 128)],
                            out_sh.at[sx_v.at[(p * 10 + o) * _JB + j]],
                            ssem, add=True)

            @pl.when(i < _NITER - 1)
            def _():
                fire_idx(i + 1)

        drain_scatters()
        plsc.subcore_barrier()
        pltpu.sync_copy(out_sh.at[pl.ds(f0, _NF // 16)],
                        out_hbm.at[pl.ds(cid * _NF + f0, _NF // 16)])

    return conv


def _make_conv10(c_out):
    nfo = _NPAD * c_out
    nhw = _NPAD * 5

    @functools.partial(
        pl.kernel,
        out_type=jax.ShapeDtypeStruct((2 * nfo,), jnp.float32),
        mesh=_MESH,
        compiler_params=_CP,
        scratch_types=[
            pltpu.VMEM((_K * 10 * c_out,), jnp.float32),
            pltpu.VMEM((2 * _JB, 128), jnp.int32),
            pltpu.VMEM((2 * _JB, 128), jnp.int32),
            pltpu.VMEM((2 * _JB, 128), jnp.int32),
            pltpu.VMEM((2 * 5 * _JB, 128), jnp.int32),
            pltpu.VMEM((2 * c_out * _JB, 128), jnp.int32),
            pltpu.VMEM((2 * 5, _B), jnp.int32),
            pltpu.VMEM((2 * c_out, _B), jnp.float32),
            pltpu.VMEM_SHARED((nhw,), jnp.int32),
            pltpu.VMEM_SHARED((nfo,), jnp.float32),
            pltpu.SemaphoreType.DMA,
            pltpu.SemaphoreType.DMA,
            pltpu.SemaphoreType.DMA,
            pltpu.SemaphoreType.DMA,
        ],
    )
    def conv(h_hbm, gi_hbm, si_hbm, of_hbm, w_hbm, z_hbm, out_hbm,
             w_v, gi_v, si_v, of_v, gx_v, sx_v, hg_v, msg_v, h_sh, out_sh,
             isem, gsem0, gsem1, ssem):
        cid = lax.axis_index("c")
        sid = lax.axis_index("s")
        wid = sid * 2 + cid
        f0 = sid * (nhw // 16)
        o0 = sid * (nfo // 16)
        pltpu.sync_copy(h_hbm.at[pl.ds(f0, nhw // 16)],
                        h_sh.at[pl.ds(f0, nhw // 16)])
        pltpu.sync_copy(z_hbm.at[pl.ds(o0, nfo // 16)],
                        out_sh.at[pl.ds(o0, nfo // 16)])
        pltpu.sync_copy(w_hbm, w_v)
        plsc.subcore_barrier()

        row0 = wid * _RPW
        rpi = 2 * _JB
        gsems = (gsem0, gsem1)

        def fire_idx(i):
            rr = row0 + i * rpi
            pltpu.async_copy(gi_hbm.at[pl.ds(rr, rpi)], gi_v, isem)
            pltpu.async_copy(si_hbm.at[pl.ds(rr, rpi)], si_v, isem)
            pltpu.async_copy(of_hbm.at[pl.ds(rr, rpi)], of_v, isem)

        def drain_scatters():
            for r in range(2 * c_out):
                for j in range(_JB):
                    pltpu.make_async_copy(
                        msg_v.at[r].at[pl.ds(j * 128, 128)],
                        out_sh.at[sx_v.at[r * _JB + j]], ssem).wait()

        fire_idx(0)

        @pl.loop(0, _NITER)
        def _step(i):
            @pl.when(i > 0)
            def _():
                drain_scatters()

            rr = row0 + i * rpi
            pltpu.make_async_copy(gi_hbm.at[pl.ds(rr, rpi)], gi_v, isem).wait()
            pltpu.make_async_copy(si_hbm.at[pl.ds(rr, rpi)], si_v, isem).wait()
            pltpu.make_async_copy(of_hbm.at[pl.ds(rr, rpi)], of_v, isem).wait()

            gps = [[], []]
            for p in range(2):
                for j in range(_JB):
                    for l in range(8):
                        sl = pl.ds(l * 16, 16)
                        s16 = gi_v[p * _JB + j, sl]
                        for k in range(5):
                            gx_v[(p * 5 + k) * _JB + j, sl] = s16 + k * _NPAD
                        d16 = si_v[p * _JB + j, sl]
                        for o in range(c_out):
                            sx_v[(p * c_out + o) * _JB + j, sl] = (
                                d16 + o * _NPAD)
                gps[p] = [
                    pltpu.async_copy(h_sh.at[gx_v.at[(p * 5 + k) * _JB + j]],
                                     hg_v.at[p * 5 + k].at[pl.ds(j * 128,
                                                                 128)],
                                     gsems[p])
                    for k in range(5) for j in range(_JB)]

            for p in range(2):
                for cp in gps[p]:
                    cp.wait()

                @pl.loop(0, _B // 16)
                def _grp(g):
                    sl = pl.ds(g * 16, 16)
                    off16 = of_v[p * _JB + g // 8, pl.ds((g % 8) * 16, 16)]
                    wb = off16 * (10 * c_out)
                    hcol = []
                    for k in range(5):
                        w16v = hg_v[p * 5 + k, sl]
                        hcol.append(plsc.bitcast(w16v << 16, jnp.float32))
                        hcol.append(plsc.bitcast(
                            w16v & jnp.int32(-65536), jnp.float32))
                    for o in range(c_out):
                        acc = hcol[0] * plsc.load_gather(w_v, [wb + o])
                        for c2 in range(1, 10):
                            acc = acc + hcol[c2] * plsc.load_gather(
                                w_v, [wb + (c2 * c_out + o)])
                        msg_v[p * c_out + o, sl] = acc

                for o in range(c_out):
                    for j in range(_JB):
                        pltpu.async_copy(
                            msg_v.at[p * c_out + o].at[pl.ds(j * 128, 128)],
                            out_sh.at[sx_v.at[(p * c_out + o) * _JB + j]],
                            ssem, add=True)

            @pl.when(i < _NITER - 1)
            def _():
                fire_idx(i + 1)

        drain_scatters()
        plsc.subcore_barrier()
        pltpu.sync_copy(out_sh.at[pl.ds(o0, nfo // 16)],
                        out_hbm.at[pl.ds(cid * nfo + o0, nfo // 16)])

    return conv


_conv1 = _make_conv1()
_conv10_10 = _make_conv10(10)
_conv10_1 = _make_conv10(1)


def _pack_bf16_pairs(h):
    hu = jax.lax.bitcast_convert_type(
        h.astype(jnp.bfloat16), jnp.uint16).astype(jnp.uint32)
    words = hu[0::2] | (hu[1::2] << 16)
    return jax.lax.bitcast_convert_type(words, jnp.int32).reshape(-1)


def kernel(x, edge_index, edge_offset, W1, b1, W2, b2, W3, b3):
    h0 = jax.nn.relu(x[:, 0].astype(jnp.float32))
    h0p = jnp.pad(h0, (0, _NPAD - _N))
    src = edge_index[0].astype(jnp.int32)
    dst = edge_index[1].astype(jnp.int32)
    off = edge_offset.astype(jnp.int32)
    npad_e = _EPAD - _E
    srcp = jnp.concatenate(
        [src, jnp.full((npad_e,), _N, jnp.int32)]).reshape(_EROWS, 128)
    dstp = jnp.concatenate(
        [dst, jnp.full((npad_e,), _N, jnp.int32)]).reshape(_EROWS, 128)
    offp = jnp.concatenate(
        [off, jnp.zeros((npad_e,), jnp.int32)]).reshape(_EROWS, 128)
    offm = (_K - 1) - offp
    z10 = jnp.zeros((_NF,), jnp.float32)
    z1 = jnp.zeros((_NPAD,), jnp.float32)
    w1f = W1.reshape(-1).astype(jnp.float32)
    w2f = W2.reshape(-1).astype(jnp.float32)
    w3f = W3.reshape(-1).astype(jnp.float32)

    def stack(gi, si, of):
        o1 = _conv1(h0p, gi, si, of, w1f, z10)
        h1 = jax.nn.relu((o1[:_NF] + o1[_NF:]).reshape(10, _NPAD) + b1[:, None])
        o2 = _conv10_10(_pack_bf16_pairs(h1), gi, si, of, w2f, z10)
        h2 = jax.nn.relu((o2[:_NF] + o2[_NF:]).reshape(10, _NPAD) + b2[:, None])
        o3 = _conv10_1(_pack_bf16_pairs(h2), gi, si, of, w3f, z1)
        return jax.nn.relu(o3[:_NPAD] + o3[_NPAD:] + b3)

    out_f = stack(srcp, dstp, offp)
    out_b = stack(dstp, srcp, offm)
    return (out_f + out_b)[:_N, None]
